# baseline (speedup 1.0000x reference)
import functools

import jax
import jax.numpy as jnp
from jax import lax
from jax.experimental import pallas as pl
from jax.experimental.pallas import tpu as pltpu
from jax.experimental.pallas import tpu_sc as plsc

_B, _V, _D, _C = 16384, 1000000, 64, 128
_H = 1 << 19
_W2 = _V - _H
_NC, _NS = 2, 16
_NW = _NC * _NS
_BPW = _B // _NW
_CH = 128
_NK = _BPW // _CH

_mesh = plsc.VectorSubcoreMesh(
    core_axis_name="c", subcore_axis_name="s", num_cores=_NC, num_subcores=_NS
)


def _worker_id():
  return lax.axis_index("s") * _NC + lax.axis_index("c")


_BP = 1024
_NBL = _H // _BP
_NBT = pl.cdiv(_V, _BP)


def _cin_body(x1_ref, x2_ref, o_ref):
  o_ref[:, 0:_D] = x1_ref[...].T
  o_ref[:, _D:128] = x2_ref[...].T


_conv_in = pl.pallas_call(
    _cin_body,
    grid=(_NBL,),
    in_specs=[
        pl.BlockSpec((_D, _BP), lambda i: (0, i)),
        pl.BlockSpec((_D, _BP), lambda i: (0, jnp.minimum(_NBL + i, _NBT - 1))),
    ],
    out_specs=pl.BlockSpec((_BP, 128), lambda i: (i, 0)),
    out_shape=jax.ShapeDtypeStruct((_H, 128), jnp.float32),
)


def _cout_body(x_ref, o_ref):
  j = pl.program_id(0)
  x = x_ref[...]
  o_ref[...] = jnp.where(j >= _NBL, x[:, _D:128], x[:, 0:_D]).T


_conv_out = pl.pallas_call(
    _cout_body,
    grid=(_NBT,),
    in_specs=[
        pl.BlockSpec((_BP, 128), lambda j: (jnp.where(j >= _NBL, j - _NBL, j), 0)),
    ],
    out_specs=pl.BlockSpec((_D, _BP), lambda j: (0, j)),
    out_shape=jax.ShapeDtypeStruct((_D, _V), jnp.float32),
)


def _gather_body(t2_hbm, idx_hbm, out_hbm, idx_v, rows_v, sem):
  wid = _worker_id()
  for k in range(_NK):
    base = (wid * _NK + k) * _CH
    pltpu.sync_copy(idx_hbm.at[pl.ds(base, _CH)], idx_v)
    pltpu.async_copy(t2_hbm.at[idx_v], rows_v, sem).wait()
    pltpu.sync_copy(rows_v, out_hbm.at[pl.ds(base, _CH)])


_gather = functools.partial(
    pl.kernel,
    out_type=jax.ShapeDtypeStruct((_B, 128), jnp.float32),
    mesh=_mesh,
    scratch_types=[
        pltpu.VMEM((_CH,), jnp.int32),
        pltpu.VMEM((_CH, 128), jnp.float32),
        pltpu.SemaphoreType.DMA,
    ],
)(_gather_body)


def _scatter_body(pay_hbm, idx_hbm, tbl_ref, idx_v, rows_v, sem):
  wid = _worker_id()
  for k in range(_NK):
    base = (wid * _NK + k) * _CH
    pltpu.sync_copy(idx_hbm.at[pl.ds(base, _CH)], idx_v)
    pltpu.sync_copy(pay_hbm.at[pl.ds(base, _CH)], rows_v)
    pltpu.async_copy(rows_v, tbl_ref.at[idx_v], sem).wait()


_scatter = functools.partial(
    pl.kernel,
    out_type=(),
    mesh=_mesh,
    scratch_types=[
        pltpu.VMEM((_CH,), jnp.int32),
        pltpu.VMEM((_CH, 128), jnp.float32),
        pltpu.SemaphoreType.DMA,
    ],
)(_scatter_body)


_BT = 2048


def _compute_body(ctx_ref, pr_ref, hf_ref, wc_ref, bc_ref, wd_ref, bd_ref, out_ref):
  ctx = ctx_ref[...]
  hf = hf_ref[...]
  emb = jnp.where(hf > 0, pr_ref[:, _D:128], pr_ref[:, 0:_D])
  ct = jax.nn.sigmoid(
      jnp.dot(ctx, wc_ref[...], preferred_element_type=jnp.float32) + bc_ref[...]
  )
  t = jnp.dot(emb, wd_ref[...], preferred_element_type=jnp.float32) + bd_ref[...]
  dl = jax.nn.sigmoid(t * ct)
  u = dl * emb + (1.0 - dl) * ct
  n = jnp.sqrt(jnp.sum(u * u, axis=1, keepdims=True))
  out_ref[...] = u / jnp.maximum(n, 1e-12)


_compute_in_specs = [
    pl.BlockSpec((_BT, _C), lambda i: (i, 0)),
    pl.BlockSpec((_BT, 128), lambda i: (i, 0)),
    pl.BlockSpec((_BT, 1), lambda i: (i, 0)),
    pl.BlockSpec((_C, _D), lambda i: (0, 0)),
    pl.BlockSpec((1, _D), lambda i: (0, 0)),
    pl.BlockSpec((_D, _D), lambda i: (0, 0)),
    pl.BlockSpec((1, _D), lambda i: (0, 0)),
]

_compute = pl.pallas_call(
    _compute_body,
    grid=(_B // _BT,),
    in_specs=_compute_in_specs,
    out_specs=pl.BlockSpec((_BT, _D), lambda i: (i, 0)),
    out_shape=jax.ShapeDtypeStruct((_B, _D), jnp.float32),
)


def _routing(flat):
  iota = jnp.arange(_B, dtype=jnp.int32)
  order = jnp.argsort(flat, stable=True)
  sv = flat[order]
  last_flag = jnp.concatenate([sv[1:] != sv[:-1], jnp.ones((1,), jnp.bool_)])
  lastpos = jnp.flip(lax.cummin(jnp.flip(jnp.where(last_flag, iota, _B))))
  winner_sorted = order[lastpos]
  inv = jnp.argsort(order, stable=True)
  src = winner_sorted[inv].astype(jnp.int32)

  half = flat >= _H
  pv = jnp.where(half, flat - _H, flat + _H)
  pos = jnp.clip(jnp.searchsorted(sv, pv), 0, _B - 1).astype(jnp.int32)
  pfound = (half | (flat < _W2)) & (sv[pos] == pv)
  psrc = winner_sorted[pos].astype(jnp.int32)
  return src, half, pfound, psrc


def kernel(inputs, context, table, W_ctx, b_ctx, W_delta, b_delta):
  flat = inputs.reshape(_B).astype(jnp.int32)
  src, half, pfound, psrc = _routing(flat)
  u_idx = jnp.where(half, flat - _H, flat)
  hf = half.astype(jnp.float32)[:, None]

  tT = table.T
  t2 = _conv_in(tT, tT)
  pairs = _gather(t2, u_idx)
  out = _compute(
      context, pairs, hf, W_ctx, b_ctx.reshape(1, _D), W_delta, b_delta.reshape(1, _D)
  )

  own = out[src]
  po = out[jnp.where(pfound, psrc, 0)]
  old_other = jnp.where(hf > 0, pairs[:, 0:_D], pairs[:, _D:128])
  other = jnp.where(pfound[:, None], po, old_other)
  lo = jnp.where(hf > 0, other, own)
  hi = jnp.where(hf > 0, own, other)
  payload = jnp.concatenate([lo, hi], axis=1)

  tref = jax.new_ref(t2)
  _scatter(payload, u_idx, tref)
  t2n = jax.freeze(tref)
  return out, _conv_out(t2n).T


# speedup vs baseline: 1.9052x; 1.9052x over previous
import functools

import jax
import jax.numpy as jnp
from jax import lax
from jax.experimental import pallas as pl
from jax.experimental.pallas import tpu as pltpu
from jax.experimental.pallas import tpu_sc as plsc

_B, _V, _D, _C = 16384, 1000000, 64, 128
_H = 1 << 19
_NC, _NS = 2, 16
_NW = _NC * _NS
_BPW = _B // _NW
_CH = 128
_NK = _BPW // _CH

_mesh = plsc.VectorSubcoreMesh(
    core_axis_name="c", subcore_axis_name="s", num_cores=_NC, num_subcores=_NS
)


def _worker_id():
  return lax.axis_index("s") * _NC + lax.axis_index("c")


_BP = 1024
_NBL = _H // _BP
_NBT = pl.cdiv(_V, _BP)


def _cin_body(x1_ref, x2_ref, o_ref):
  o_ref[:, 0:_D] = x1_ref[...].T
  o_ref[:, _D:128] = x2_ref[...].T


_conv_in = pl.pallas_call(
    _cin_body,
    grid=(_NBL,),
    in_specs=[
        pl.BlockSpec((_D, _BP), lambda i: (0, i)),
        pl.BlockSpec((_D, _BP), lambda i: (0, jnp.minimum(_NBL + i, _NBT - 1))),
    ],
    out_specs=pl.BlockSpec((_BP, 128), lambda i: (i, 0)),
    out_shape=jax.ShapeDtypeStruct((_H, 128), jnp.float32),
)


def _cout_body(x_ref, o_ref):
  j = pl.program_id(0)
  xt = x_ref[...].T
  o_ref[...] = jnp.where(j >= _NBL, xt[_D:128, :], xt[0:_D, :])


_conv_out = pl.pallas_call(
    _cout_body,
    grid=(_NBT,),
    in_specs=[
        pl.BlockSpec((_BP, 128), lambda j: (jnp.where(j >= _NBL, j - _NBL, j), 0)),
    ],
    out_specs=pl.BlockSpec((_D, _BP), lambda j: (0, j)),
    out_shape=jax.ShapeDtypeStruct((_D, _V), jnp.float32),
)


_SC_PARAMS = pltpu.CompilerParams(use_tc_tiling_on_sc=False)


def _gather_body(t2f_hbm, idx_hbm, out_hbm, idx_v, rows_v, sem):
  wid = _worker_id()
  for k in range(_NK):
    base = (wid * _NK + k) * _CH
    pltpu.sync_copy(idx_hbm.at[pl.ds(base, _CH)], idx_v)
    pltpu.async_copy(t2f_hbm.at[idx_v], rows_v, sem).wait()
    pltpu.sync_copy(rows_v, out_hbm.at[pl.ds(base, _CH)])


_gather = functools.partial(
    pl.kernel,
    out_type=jax.ShapeDtypeStruct((_B, _D), jnp.float32),
    mesh=_mesh,
    scratch_types=[
        pltpu.VMEM((_CH,), jnp.int32),
        pltpu.VMEM((_CH, _D), jnp.float32),
        pltpu.SemaphoreType.DMA,
    ],
    compiler_params=_SC_PARAMS,
)(_gather_body)


def _scatter_body(upd_hbm, idx_hbm, src_hbm, tbl_ref, idx_v, src_v, rows_v, sem):
  wid = _worker_id()
  for k in range(_NK):
    base = (wid * _NK + k) * _CH
    pltpu.sync_copy(idx_hbm.at[pl.ds(base, _CH)], idx_v)
    pltpu.sync_copy(src_hbm.at[pl.ds(base, _CH)], src_v)
    pltpu.async_copy(upd_hbm.at[src_v], rows_v, sem).wait()
    pltpu.async_copy(rows_v, tbl_ref.at[idx_v], sem).wait()


_scatter = functools.partial(
    pl.kernel,
    out_type=(),
    mesh=_mesh,
    scratch_types=[
        pltpu.VMEM((_CH,), jnp.int32),
        pltpu.VMEM((_CH,), jnp.int32),
        pltpu.VMEM((_CH, _D), jnp.float32),
        pltpu.SemaphoreType.DMA,
    ],
    compiler_params=_SC_PARAMS,
)(_scatter_body)


_BT = 2048


def _compute_body(ctx_ref, emb_ref, wc_ref, bc_ref, wd_ref, bd_ref, out_ref):
  ctx = ctx_ref[...]
  emb = emb_ref[...]
  ct = jax.nn.sigmoid(
      jnp.dot(ctx, wc_ref[...], preferred_element_type=jnp.float32) + bc_ref[...]
  )
  t = jnp.dot(emb, wd_ref[...], preferred_element_type=jnp.float32) + bd_ref[...]
  dl = jax.nn.sigmoid(t * ct)
  u = dl * emb + (1.0 - dl) * ct
  n = jnp.sqrt(jnp.sum(u * u, axis=1, keepdims=True))
  out_ref[...] = u / jnp.maximum(n, 1e-12)


_compute_in_specs = [
    pl.BlockSpec((_BT, _C), lambda i: (i, 0)),
    pl.BlockSpec((_BT, _D), lambda i: (i, 0)),
    pl.BlockSpec((_C, _D), lambda i: (0, 0)),
    pl.BlockSpec((1, _D), lambda i: (0, 0)),
    pl.BlockSpec((_D, _D), lambda i: (0, 0)),
    pl.BlockSpec((1, _D), lambda i: (0, 0)),
]

_compute = pl.pallas_call(
    _compute_body,
    grid=(_B // _BT,),
    in_specs=_compute_in_specs,
    out_specs=pl.BlockSpec((_BT, _D), lambda i: (i, 0)),
    out_shape=jax.ShapeDtypeStruct((_B, _D), jnp.float32),
)


def _winner_src(flat):
  iota = jnp.arange(_B, dtype=jnp.int32)
  order = jnp.argsort(flat, stable=True)
  sv = flat[order]
  last_flag = jnp.concatenate([sv[1:] != sv[:-1], jnp.ones((1,), jnp.bool_)])
  lastpos = jnp.flip(lax.cummin(jnp.flip(jnp.where(last_flag, iota, _B))))
  winner_sorted = order[lastpos]
  inv = jnp.argsort(order, stable=True)
  return winner_sorted[inv].astype(jnp.int32)


def kernel(inputs, context, table, W_ctx, b_ctx, W_delta, b_delta):
  flat = inputs.reshape(_B).astype(jnp.int32)
  src = _winner_src(flat)
  half = flat >= _H
  fr = jnp.where(half, 2 * (flat - _H) + 1, 2 * flat)

  tT = table.T
  t2f = _conv_in(tT, tT).reshape(2 * _H, _D)
  emb = _gather(t2f, fr)
  out = _compute(
      context, emb, W_ctx, b_ctx.reshape(1, _D), W_delta, b_delta.reshape(1, _D)
  )

  tref = jax.new_ref(t2f)
  _scatter(out, fr, src, tref)
  t2n = jax.freeze(tref).reshape(_H, 128)
  return out, _conv_out(t2n).T


# speedup vs baseline: 2.7003x; 1.4173x over previous
import functools

import jax
import jax.numpy as jnp
from jax import lax
from jax.experimental import pallas as pl
from jax.experimental.pallas import tpu as pltpu
from jax.experimental.pallas import tpu_sc as plsc

_B, _V, _D, _C = 16384, 1000000, 64, 128
_H = 1 << 19
_NC, _NS = 2, 16
_NW = _NC * _NS
_BPW = _B // _NW
_CH = 128
_NK = _BPW // _CH

_mesh = plsc.VectorSubcoreMesh(
    core_axis_name="c", subcore_axis_name="s", num_cores=_NC, num_subcores=_NS
)


def _worker_id():
  return lax.axis_index("s") * _NC + lax.axis_index("c")


_BP = 2048
_NBL = _H // _BP
_NBT = pl.cdiv(_V, _BP)


def _cin_body(x1_ref, x2_ref, o_ref):
  o_ref[:, 0:_D] = x1_ref[...].T
  o_ref[:, _D:128] = x2_ref[...].T


_conv_in = pl.pallas_call(
    _cin_body,
    grid=(_NBL,),
    in_specs=[
        pl.BlockSpec((_D, _BP), lambda i: (0, i)),
        pl.BlockSpec((_D, _BP), lambda i: (0, jnp.minimum(_NBL + i, _NBT - 1))),
    ],
    out_specs=pl.BlockSpec((_BP, 128), lambda i: (i, 0)),
    out_shape=jax.ShapeDtypeStruct((_H, 128), jnp.float32),
)


def _cout_body(x_ref, o_ref):
  j = pl.program_id(0)
  xt = x_ref[...].T
  o_ref[...] = jnp.where(j >= _NBL, xt[_D:128, :], xt[0:_D, :])


_conv_out = pl.pallas_call(
    _cout_body,
    grid=(_NBT,),
    in_specs=[
        pl.BlockSpec((_BP, 128), lambda j: (jnp.where(j >= _NBL, j - _NBL, j), 0)),
    ],
    out_specs=pl.BlockSpec((_D, _BP), lambda j: (0, j)),
    out_shape=jax.ShapeDtypeStruct((_D, _V), jnp.float32),
)


_SC_PARAMS = pltpu.CompilerParams(use_tc_tiling_on_sc=False)


def _gather_body(t2f_hbm, idx_hbm, out_hbm, idx_v, rows_v, sem):
  wid = _worker_id()
  for k in range(_NK):
    base = (wid * _NK + k) * _CH
    pltpu.sync_copy(idx_hbm.at[pl.ds(base, _CH)], idx_v)
    pltpu.async_copy(t2f_hbm.at[idx_v], rows_v, sem).wait()
    pltpu.sync_copy(rows_v, out_hbm.at[pl.ds(base, _CH)])


_gather = functools.partial(
    pl.kernel,
    out_type=jax.ShapeDtypeStruct((_B, _D), jnp.float32),
    mesh=_mesh,
    scratch_types=[
        pltpu.VMEM((_CH,), jnp.int32),
        pltpu.VMEM((_CH, _D), jnp.float32),
        pltpu.SemaphoreType.DMA,
    ],
    compiler_params=_SC_PARAMS,
)(_gather_body)


def _scatter_body(upd_hbm, idx_hbm, src_hbm, tbl_ref, idx_v, src_v, rows_v, sem):
  wid = _worker_id()
  for k in range(_NK):
    base = (wid * _NK + k) * _CH
    pltpu.sync_copy(idx_hbm.at[pl.ds(base, _CH)], idx_v)
    pltpu.sync_copy(src_hbm.at[pl.ds(base, _CH)], src_v)
    pltpu.async_copy(upd_hbm.at[src_v], rows_v, sem).wait()
    pltpu.async_copy(rows_v, tbl_ref.at[idx_v], sem).wait()


_scatter = functools.partial(
    pl.kernel,
    out_type=(),
    mesh=_mesh,
    scratch_types=[
        pltpu.VMEM((_CH,), jnp.int32),
        pltpu.VMEM((_CH,), jnp.int32),
        pltpu.VMEM((_CH, _D), jnp.float32),
        pltpu.SemaphoreType.DMA,
    ],
    compiler_params=_SC_PARAMS,
)(_scatter_body)


_BT = 2048


def _compute_body(ctx_ref, emb_ref, wc_ref, bc_ref, wd_ref, bd_ref, out_ref):
  ctx = ctx_ref[...]
  emb = emb_ref[...]
  ct = jax.nn.sigmoid(
      jnp.dot(ctx, wc_ref[...], preferred_element_type=jnp.float32) + bc_ref[...]
  )
  t = jnp.dot(emb, wd_ref[...], preferred_element_type=jnp.float32) + bd_ref[...]
  dl = jax.nn.sigmoid(t * ct)
  u = dl * emb + (1.0 - dl) * ct
  n = jnp.sqrt(jnp.sum(u * u, axis=1, keepdims=True))
  out_ref[...] = u / jnp.maximum(n, 1e-12)


_compute_in_specs = [
    pl.BlockSpec((_BT, _C), lambda i: (i, 0)),
    pl.BlockSpec((_BT, _D), lambda i: (i, 0)),
    pl.BlockSpec((_C, _D), lambda i: (0, 0)),
    pl.BlockSpec((1, _D), lambda i: (0, 0)),
    pl.BlockSpec((_D, _D), lambda i: (0, 0)),
    pl.BlockSpec((1, _D), lambda i: (0, 0)),
]

_compute = pl.pallas_call(
    _compute_body,
    grid=(_B // _BT,),
    in_specs=_compute_in_specs,
    out_specs=pl.BlockSpec((_BT, _D), lambda i: (i, 0)),
    out_shape=jax.ShapeDtypeStruct((_B, _D), jnp.float32),
)


def _winner_src(flat):
  iota = jnp.arange(_B, dtype=jnp.int32)
  order = jnp.argsort(flat, stable=True)
  sv = flat[order]
  last_flag = jnp.concatenate([sv[1:] != sv[:-1], jnp.ones((1,), jnp.bool_)])
  lastpos = jnp.flip(lax.cummin(jnp.flip(jnp.where(last_flag, iota, _B))))
  winner_sorted = order[lastpos]
  inv = jnp.argsort(order, stable=True)
  return winner_sorted[inv].astype(jnp.int32)


def kernel(inputs, context, table, W_ctx, b_ctx, W_delta, b_delta):
  flat = inputs.reshape(_B).astype(jnp.int32)
  src = _winner_src(flat)
  half = flat >= _H
  fr = jnp.where(half, 2 * (flat - _H) + 1, 2 * flat)

  tT = table.T
  t2f = _conv_in(tT, tT).reshape(2 * _H, _D)
  emb = _gather(t2f, fr)
  out = _compute(
      context, emb, W_ctx, b_ctx.reshape(1, _D), W_delta, b_delta.reshape(1, _D)
  )

  tref = jax.new_ref(t2f)
  _scatter(out, fr, src, tref)
  t2n = jax.freeze(tref).reshape(_H, 128)
  return out, _conv_out(t2n).T


# speedup vs baseline: 3.4658x; 1.2835x over previous
import functools

import jax
import jax.numpy as jnp
from jax import lax
from jax.experimental import pallas as pl
from jax.experimental.pallas import tpu as pltpu
from jax.experimental.pallas import tpu_sc as plsc

_B, _V, _D, _C = 16384, 1000000, 64, 128
_H = 1 << 19
_NC, _NS = 2, 16
_NW = _NC * _NS
_BPW = _B // _NW
_CH = 128
_NK = _BPW // _CH

_mesh = plsc.VectorSubcoreMesh(
    core_axis_name="c", subcore_axis_name="s", num_cores=_NC, num_subcores=_NS
)


def _worker_id():
  return lax.axis_index("s") * _NC + lax.axis_index("c")


_BP = 4096
_NBL = _H // _BP
_NBT = pl.cdiv(_V, _BP)


def _cin_body(x1_ref, x2_ref, o_ref):
  o_ref[:, 0:_D] = x1_ref[...].T
  o_ref[:, _D:128] = x2_ref[...].T


_conv_in = pl.pallas_call(
    _cin_body,
    grid=(_NBL,),
    in_specs=[
        pl.BlockSpec((_D, _BP), lambda i: (0, i)),
        pl.BlockSpec((_D, _BP), lambda i: (0, jnp.minimum(_NBL + i, _NBT - 1))),
    ],
    out_specs=pl.BlockSpec((_BP, 128), lambda i: (i, 0)),
    out_shape=jax.ShapeDtypeStruct((_H, 128), jnp.float32),
)


def _cout_body(x_ref, o_ref):
  j = pl.program_id(0)
  xt = x_ref[...].T
  o_ref[...] = jnp.where(j >= _NBL, xt[_D:128, :], xt[0:_D, :])


_conv_out = pl.pallas_call(
    _cout_body,
    grid=(_NBT,),
    in_specs=[
        pl.BlockSpec((_BP, 128), lambda j: (jnp.where(j >= _NBL, j - _NBL, j), 0)),
    ],
    out_specs=pl.BlockSpec((_D, _BP), lambda j: (0, j)),
    out_shape=jax.ShapeDtypeStruct((_D, _V), jnp.float32),
)


_SC_PARAMS = pltpu.CompilerParams(use_tc_tiling_on_sc=False)


def _gather_body(t2f_hbm, idx_hbm, out_hbm, idx_v, rows_v, sem):
  wid = _worker_id()
  for k in range(_NK):
    base = (wid * _NK + k) * _CH
    pltpu.sync_copy(idx_hbm.at[pl.ds(base, _CH)], idx_v)
    pltpu.async_copy(t2f_hbm.at[idx_v], rows_v, sem).wait()
    pltpu.sync_copy(rows_v, out_hbm.at[pl.ds(base, _CH)])


_gather = functools.partial(
    pl.kernel,
    out_type=jax.ShapeDtypeStruct((_B, _D), jnp.float32),
    mesh=_mesh,
    scratch_types=[
        pltpu.VMEM((_CH,), jnp.int32),
        pltpu.VMEM((_CH, _D), jnp.float32),
        pltpu.SemaphoreType.DMA,
    ],
    compiler_params=_SC_PARAMS,
)(_gather_body)


def _scatter_body(upd_hbm, idx_hbm, src_hbm, tbl_ref, idx_v, src_v, rows_v, sem):
  wid = _worker_id()
  for k in range(_NK):
    base = (wid * _NK + k) * _CH
    pltpu.sync_copy(idx_hbm.at[pl.ds(base, _CH)], idx_v)
    pltpu.sync_copy(src_hbm.at[pl.ds(base, _CH)], src_v)
    pltpu.async_copy(upd_hbm.at[src_v], rows_v, sem).wait()
    pltpu.async_copy(rows_v, tbl_ref.at[idx_v], sem).wait()


_scatter = functools.partial(
    pl.kernel,
    out_type=(),
    mesh=_mesh,
    scratch_types=[
        pltpu.VMEM((_CH,), jnp.int32),
        pltpu.VMEM((_CH,), jnp.int32),
        pltpu.VMEM((_CH, _D), jnp.float32),
        pltpu.SemaphoreType.DMA,
    ],
    compiler_params=_SC_PARAMS,
)(_scatter_body)


_BT = 2048


def _compute_body(ctx_ref, emb_ref, wc_ref, bc_ref, wd_ref, bd_ref, out_ref):
  ctx = ctx_ref[...]
  emb = emb_ref[...]
  ct = jax.nn.sigmoid(
      jnp.dot(ctx, wc_ref[...], preferred_element_type=jnp.float32) + bc_ref[...]
  )
  t = jnp.dot(emb, wd_ref[...], preferred_element_type=jnp.float32) + bd_ref[...]
  dl = jax.nn.sigmoid(t * ct)
  u = dl * emb + (1.0 - dl) * ct
  n = jnp.sqrt(jnp.sum(u * u, axis=1, keepdims=True))
  out_ref[...] = u / jnp.maximum(n, 1e-12)


_compute_in_specs = [
    pl.BlockSpec((_BT, _C), lambda i: (i, 0)),
    pl.BlockSpec((_BT, _D), lambda i: (i, 0)),
    pl.BlockSpec((_C, _D), lambda i: (0, 0)),
    pl.BlockSpec((1, _D), lambda i: (0, 0)),
    pl.BlockSpec((_D, _D), lambda i: (0, 0)),
    pl.BlockSpec((1, _D), lambda i: (0, 0)),
]

_compute = pl.pallas_call(
    _compute_body,
    grid=(_B // _BT,),
    in_specs=_compute_in_specs,
    out_specs=pl.BlockSpec((_BT, _D), lambda i: (i, 0)),
    out_shape=jax.ShapeDtypeStruct((_B, _D), jnp.float32),
)


def _winner_src(flat):
  iota = jnp.arange(_B, dtype=jnp.int32)
  order = jnp.argsort(flat, stable=True)
  sv = flat[order]
  last_flag = jnp.concatenate([sv[1:] != sv[:-1], jnp.ones((1,), jnp.bool_)])
  lastpos = jnp.flip(lax.cummin(jnp.flip(jnp.where(last_flag, iota, _B))))
  winner_sorted = order[lastpos]
  inv = jnp.argsort(order, stable=True)
  return winner_sorted[inv].astype(jnp.int32)


def kernel(inputs, context, table, W_ctx, b_ctx, W_delta, b_delta):
  flat = inputs.reshape(_B).astype(jnp.int32)
  src = _winner_src(flat)
  half = flat >= _H
  fr = jnp.where(half, 2 * (flat - _H) + 1, 2 * flat)

  tT = table.T
  t2f = _conv_in(tT, tT).reshape(2 * _H, _D)
  emb = _gather(t2f, fr)
  out = _compute(
      context, emb, W_ctx, b_ctx.reshape(1, _D), W_delta, b_delta.reshape(1, _D)
  )

  tref = jax.new_ref(t2f)
  _scatter(out, fr, src, tref)
  t2n = jax.freeze(tref).reshape(_H, 128)
  return out, _conv_out(t2n).T


# speedup vs baseline: 4.1173x; 1.1880x over previous
import functools

import jax
import jax.numpy as jnp
from jax import lax
from jax.experimental import pallas as pl
from jax.experimental.pallas import tpu as pltpu
from jax.experimental.pallas import tpu_sc as plsc

_B, _V, _D, _C = 16384, 1000000, 64, 128
_H = 1 << 19
_NC, _NS = 2, 16
_NW = _NC * _NS
_BPW = _B // _NW
_CH = 128
_NK = _BPW // _CH

_mesh = plsc.VectorSubcoreMesh(
    core_axis_name="c", subcore_axis_name="s", num_cores=_NC, num_subcores=_NS
)


def _worker_id():
  return lax.axis_index("s") * _NC + lax.axis_index("c")


_BP = 8192
_NBL = _H // _BP
_NBT = pl.cdiv(_V, _BP)


def _cin_body(x1_ref, x2_ref, o_ref):
  o_ref[:, 0:_D] = x1_ref[...].T
  o_ref[:, _D:128] = x2_ref[...].T


_conv_in = pl.pallas_call(
    _cin_body,
    grid=(_NBL,),
    in_specs=[
        pl.BlockSpec((_D, _BP), lambda i: (0, i)),
        pl.BlockSpec((_D, _BP), lambda i: (0, jnp.minimum(_NBL + i, _NBT - 1))),
    ],
    out_specs=pl.BlockSpec((_BP, 128), lambda i: (i, 0)),
    out_shape=jax.ShapeDtypeStruct((_H, 128), jnp.float32),
)


def _cout_body(x_ref, o_ref):
  j = pl.program_id(0)
  xt = x_ref[...].T
  o_ref[...] = jnp.where(j >= _NBL, xt[_D:128, :], xt[0:_D, :])


_conv_out = pl.pallas_call(
    _cout_body,
    grid=(_NBT,),
    in_specs=[
        pl.BlockSpec((_BP, 128), lambda j: (jnp.where(j >= _NBL, j - _NBL, j), 0)),
    ],
    out_specs=pl.BlockSpec((_D, _BP), lambda j: (0, j)),
    out_shape=jax.ShapeDtypeStruct((_D, _V), jnp.float32),
)


_SC_PARAMS = pltpu.CompilerParams(use_tc_tiling_on_sc=False)


def _gather_body(t2f_hbm, idx_hbm, out_hbm, idx_v, rows_v, sem):
  wid = _worker_id()
  for k in range(_NK):
    base = (wid * _NK + k) * _CH
    pltpu.sync_copy(idx_hbm.at[pl.ds(base, _CH)], idx_v)
    pltpu.async_copy(t2f_hbm.at[idx_v], rows_v, sem).wait()
    pltpu.sync_copy(rows_v, out_hbm.at[pl.ds(base, _CH)])


_gather = functools.partial(
    pl.kernel,
    out_type=jax.ShapeDtypeStruct((_B, _D), jnp.float32),
    mesh=_mesh,
    scratch_types=[
        pltpu.VMEM((_CH,), jnp.int32),
        pltpu.VMEM((_CH, _D), jnp.float32),
        pltpu.SemaphoreType.DMA,
    ],
    compiler_params=_SC_PARAMS,
)(_gather_body)


def _scatter_body(upd_hbm, idx_hbm, src_hbm, tbl_ref, idx_v, src_v, rows_v, sem):
  wid = _worker_id()
  for k in range(_NK):
    base = (wid * _NK + k) * _CH
    pltpu.sync_copy(idx_hbm.at[pl.ds(base, _CH)], idx_v)
    pltpu.sync_copy(src_hbm.at[pl.ds(base, _CH)], src_v)
    pltpu.async_copy(upd_hbm.at[src_v], rows_v, sem).wait()
    pltpu.async_copy(rows_v, tbl_ref.at[idx_v], sem).wait()


_scatter = functools.partial(
    pl.kernel,
    out_type=(),
    mesh=_mesh,
    scratch_types=[
        pltpu.VMEM((_CH,), jnp.int32),
        pltpu.VMEM((_CH,), jnp.int32),
        pltpu.VMEM((_CH, _D), jnp.float32),
        pltpu.SemaphoreType.DMA,
    ],
    compiler_params=_SC_PARAMS,
)(_scatter_body)


_BT = 2048


def _compute_body(ctx_ref, emb_ref, wc_ref, bc_ref, wd_ref, bd_ref, out_ref):
  ctx = ctx_ref[...]
  emb = emb_ref[...]
  ct = jax.nn.sigmoid(
      jnp.dot(ctx, wc_ref[...], preferred_element_type=jnp.float32) + bc_ref[...]
  )
  t = jnp.dot(emb, wd_ref[...], preferred_element_type=jnp.float32) + bd_ref[...]
  dl = jax.nn.sigmoid(t * ct)
  u = dl * emb + (1.0 - dl) * ct
  n = jnp.sqrt(jnp.sum(u * u, axis=1, keepdims=True))
  out_ref[...] = u / jnp.maximum(n, 1e-12)


_compute_in_specs = [
    pl.BlockSpec((_BT, _C), lambda i: (i, 0)),
    pl.BlockSpec((_BT, _D), lambda i: (i, 0)),
    pl.BlockSpec((_C, _D), lambda i: (0, 0)),
    pl.BlockSpec((1, _D), lambda i: (0, 0)),
    pl.BlockSpec((_D, _D), lambda i: (0, 0)),
    pl.BlockSpec((1, _D), lambda i: (0, 0)),
]

_compute = pl.pallas_call(
    _compute_body,
    grid=(_B // _BT,),
    in_specs=_compute_in_specs,
    out_specs=pl.BlockSpec((_BT, _D), lambda i: (i, 0)),
    out_shape=jax.ShapeDtypeStruct((_B, _D), jnp.float32),
)


def _winner_src(flat):
  iota = jnp.arange(_B, dtype=jnp.int32)
  order = jnp.argsort(flat, stable=True)
  sv = flat[order]
  last_flag = jnp.concatenate([sv[1:] != sv[:-1], jnp.ones((1,), jnp.bool_)])
  lastpos = jnp.flip(lax.cummin(jnp.flip(jnp.where(last_flag, iota, _B))))
  winner_sorted = order[lastpos]
  inv = jnp.argsort(order, stable=True)
  return winner_sorted[inv].astype(jnp.int32)


def kernel(inputs, context, table, W_ctx, b_ctx, W_delta, b_delta):
  flat = inputs.reshape(_B).astype(jnp.int32)
  src = _winner_src(flat)
  half = flat >= _H
  fr = jnp.where(half, 2 * (flat - _H) + 1, 2 * flat)

  tT = table.T
  t2f = _conv_in(tT, tT).reshape(2 * _H, _D)
  emb = _gather(t2f, fr)
  out = _compute(
      context, emb, W_ctx, b_ctx.reshape(1, _D), W_delta, b_delta.reshape(1, _D)
  )

  tref = jax.new_ref(t2f)
  _scatter(out, fr, src, tref)
  t2n = jax.freeze(tref).reshape(_H, 128)
  return out, _conv_out(t2n).T
